# baseline (device time: 58802 ns/iter reference)
import jax
import jax.numpy as jnp
from jax import lax
from jax.experimental import pallas as pl
from jax.experimental.pallas import tpu as pltpu


def kernel(x, pi):
    def body(pi_ref, x_ref, out_ref, send_sem, recv_sem):
        my_x = lax.axis_index("x")
        my_y = lax.axis_index("y")
        my_z = lax.axis_index("z")
        dst_x = pi_ref[my_x]

        @pl.when(dst_x == my_x)
        def _identity():
            out_ref[...] = x_ref[...]

        @pl.when(dst_x != my_x)
        def _swap():
            rdma = pltpu.make_async_remote_copy(
                src_ref=x_ref,
                dst_ref=out_ref,
                send_sem=send_sem,
                recv_sem=recv_sem,
                device_id=(dst_x, my_y, my_z),
                device_id_type=pl.DeviceIdType.MESH,
            )
            rdma.start()
            rdma.wait()

    return pl.pallas_call(
        body,
        out_shape=jax.ShapeDtypeStruct(x.shape, x.dtype),
        in_specs=[
            pl.BlockSpec(memory_space=pltpu.SMEM),
            pl.BlockSpec(memory_space=pltpu.VMEM),
        ],
        out_specs=pl.BlockSpec(memory_space=pltpu.VMEM),
        scratch_shapes=[
            pltpu.SemaphoreType.DMA,
            pltpu.SemaphoreType.DMA,
        ],
    )(pi, x)


# device time: 54002 ns/iter; 1.0889x vs baseline; 1.0889x over previous
import jax
import jax.numpy as jnp
from jax import lax
from jax.experimental import pallas as pl
from jax.experimental.pallas import tpu as pltpu


def kernel(x, pi):
    def body(pi_ref, x_ref, out_ref, send_sem, recv_sem):
        my_x = lax.axis_index("x")
        my_y = lax.axis_index("y")
        my_z = lax.axis_index("z")
        dst_x = pi_ref[my_x]

        barrier_sem = pltpu.get_barrier_semaphore()
        pl.semaphore_signal(
            barrier_sem,
            inc=1,
            device_id=(1 - my_x, my_y, my_z),
            device_id_type=pl.DeviceIdType.MESH,
        )
        pl.semaphore_wait(barrier_sem, 1)

        @pl.when(dst_x == my_x)
        def _identity():
            out_ref[...] = x_ref[...]

        @pl.when(dst_x != my_x)
        def _swap():
            rdma = pltpu.make_async_remote_copy(
                src_ref=x_ref,
                dst_ref=out_ref,
                send_sem=send_sem,
                recv_sem=recv_sem,
                device_id=(dst_x, my_y, my_z),
                device_id_type=pl.DeviceIdType.MESH,
            )
            rdma.start()
            rdma.wait()

    return pl.pallas_call(
        body,
        out_shape=jax.ShapeDtypeStruct(x.shape, x.dtype),
        in_specs=[
            pl.BlockSpec(memory_space=pltpu.SMEM),
            pl.BlockSpec(memory_space=pltpu.VMEM),
        ],
        out_specs=pl.BlockSpec(memory_space=pltpu.VMEM),
        scratch_shapes=[
            pltpu.SemaphoreType.DMA,
            pltpu.SemaphoreType.DMA,
        ],
        compiler_params=pltpu.CompilerParams(collective_id=0),
    )(pi, x)


# device time: 33609 ns/iter; 1.7496x vs baseline; 1.6068x over previous
import jax
import jax.numpy as jnp
from jax import lax
from jax.experimental import pallas as pl
from jax.experimental.pallas import tpu as pltpu

QROWS = 256
CH = 64
NCH = QROWS // CH
NMSG = 3 * NCH + 2


def kernel(x, pi):
    def body(pi_ref, x_ref, out_ref, send_sems, recv_sems):
        my_x = lax.axis_index("x")
        my_y = lax.axis_index("y")
        my_z = lax.axis_index("z")
        dst_x = pi_ref[my_x]

        @pl.when(dst_x == my_x)
        def _identity():
            out_ref[...] = x_ref[...]

        @pl.when(dst_x != my_x)
        def _swap():
            even = (my_y + my_z) % 2 == 0
            p = jnp.where(my_y == 0, my_z, 3 - my_z)
            cw = (my_x, jnp.where(even, my_y, 1 - my_y),
                  jnp.where(even, 1 - my_z, my_z))
            ccw = (my_x, jnp.where(even, 1 - my_y, my_y),
                   jnp.where(even, my_z, 1 - my_z))
            part = (dst_x, my_y, my_z)

            barrier_sem = pltpu.get_barrier_semaphore()
            for nbr in (part, cw, ccw):
                pl.semaphore_signal(
                    barrier_sem, inc=1, device_id=nbr,
                    device_id_type=pl.DeviceIdType.MESH,
                )
            pl.semaphore_wait(barrier_sem, 3)

            bp = p * QROWS
            b_cw = ((p + 1) % 4) * QROWS
            b_ccw = ((p + 3) % 4) * QROWS

            def rdma(rows, nrows, msg, dev):
                sl = (slice(None), pl.ds(rows, nrows), slice(None))
                src = x_ref.at[sl] if msg < NCH else out_ref.at[sl]
                return pltpu.make_async_remote_copy(
                    src_ref=src,
                    dst_ref=out_ref.at[sl],
                    send_sem=send_sems.at[msg],
                    recv_sem=recv_sems.at[msg],
                    device_id=dev,
                    device_id_type=pl.DeviceIdType.MESH,
                )

            xs = [rdma(bp + k * CH, CH, k, part) for k in range(NCH)]
            for d in xs:
                d.start()

            bcw, bccw = [], []
            for k in range(NCH):
                xs[k].wait_recv()
                d1 = rdma(bp + k * CH, CH, NCH + k, cw)
                d2 = rdma(bp + k * CH, CH, 2 * NCH + k, ccw)
                d1.start()
                d2.start()
                bcw.append(d1)
                bccw.append(d2)

            for k in range(NCH // 2):
                bccw[k].wait_recv()
            f_ccw = rdma(b_cw, QROWS // 2, 3 * NCH + 1, ccw)
            f_ccw.start()
            for k in range(NCH // 2, NCH):
                bcw[k].wait_recv()
            f_cw = rdma(b_ccw + QROWS // 2, QROWS // 2, 3 * NCH, cw)
            f_cw.start()

            for k in range(NCH // 2):
                bcw[k].wait_recv()
            for k in range(NCH // 2, NCH):
                bccw[k].wait_recv()
            f_ccw.wait_recv()
            f_cw.wait_recv()
            for d in xs + bcw + bccw + [f_cw, f_ccw]:
                d.wait_send()

    return pl.pallas_call(
        body,
        out_shape=jax.ShapeDtypeStruct(x.shape, x.dtype),
        in_specs=[
            pl.BlockSpec(memory_space=pltpu.SMEM),
            pl.BlockSpec(memory_space=pltpu.VMEM),
        ],
        out_specs=pl.BlockSpec(memory_space=pltpu.VMEM),
        scratch_shapes=[
            pltpu.SemaphoreType.DMA((NMSG,)),
            pltpu.SemaphoreType.DMA((NMSG,)),
        ],
        compiler_params=pltpu.CompilerParams(collective_id=0),
    )(pi, x)


# device time: 32461 ns/iter; 1.8115x vs baseline; 1.0354x over previous
import jax
import jax.numpy as jnp
from jax import lax
from jax.experimental import pallas as pl
from jax.experimental.pallas import tpu as pltpu

QROWS = 256
CH = 32
NCH = QROWS // CH
NMSG = 3 * NCH + 2


def kernel(x, pi):
    def body(pi_ref, x_ref, out_ref, send_sems, recv_sems):
        my_x = lax.axis_index("x")
        my_y = lax.axis_index("y")
        my_z = lax.axis_index("z")
        dst_x = pi_ref[my_x]

        @pl.when(dst_x == my_x)
        def _identity():
            out_ref[...] = x_ref[...]

        @pl.when(dst_x != my_x)
        def _swap():
            even = (my_y + my_z) % 2 == 0
            p = jnp.where(my_y == 0, my_z, 3 - my_z)
            cw = (my_x, jnp.where(even, my_y, 1 - my_y),
                  jnp.where(even, 1 - my_z, my_z))
            ccw = (my_x, jnp.where(even, 1 - my_y, my_y),
                   jnp.where(even, my_z, 1 - my_z))
            part = (dst_x, my_y, my_z)

            barrier_sem = pltpu.get_barrier_semaphore()
            for nbr in (part, cw, ccw):
                pl.semaphore_signal(
                    barrier_sem, inc=1, device_id=nbr,
                    device_id_type=pl.DeviceIdType.MESH,
                )
            pl.semaphore_wait(barrier_sem, 3)

            bp = p * QROWS
            b_cw = ((p + 1) % 4) * QROWS
            b_ccw = ((p + 3) % 4) * QROWS

            def rdma(rows, nrows, msg, dev):
                sl = (slice(None), pl.ds(rows, nrows), slice(None))
                src = x_ref.at[sl] if msg < NCH else out_ref.at[sl]
                return pltpu.make_async_remote_copy(
                    src_ref=src,
                    dst_ref=out_ref.at[sl],
                    send_sem=send_sems.at[msg],
                    recv_sem=recv_sems.at[msg],
                    device_id=dev,
                    device_id_type=pl.DeviceIdType.MESH,
                )

            xs = [rdma(bp + k * CH, CH, k, part) for k in range(NCH)]
            for d in xs:
                d.start()

            bcw, bccw = [], []
            for k in range(NCH):
                xs[k].wait_recv()
                d1 = rdma(bp + k * CH, CH, NCH + k, cw)
                d2 = rdma(bp + k * CH, CH, 2 * NCH + k, ccw)
                d1.start()
                d2.start()
                bcw.append(d1)
                bccw.append(d2)

            for k in range(NCH // 2):
                bccw[k].wait_recv()
            f_ccw = rdma(b_cw, QROWS // 2, 3 * NCH + 1, ccw)
            f_ccw.start()
            for k in range(NCH // 2, NCH):
                bcw[k].wait_recv()
            f_cw = rdma(b_ccw + QROWS // 2, QROWS // 2, 3 * NCH, cw)
            f_cw.start()

            for k in range(NCH // 2):
                bcw[k].wait_recv()
            for k in range(NCH // 2, NCH):
                bccw[k].wait_recv()
            f_ccw.wait_recv()
            f_cw.wait_recv()
            for d in xs + bcw + bccw + [f_cw, f_ccw]:
                d.wait_send()

    return pl.pallas_call(
        body,
        out_shape=jax.ShapeDtypeStruct(x.shape, x.dtype),
        in_specs=[
            pl.BlockSpec(memory_space=pltpu.SMEM),
            pl.BlockSpec(memory_space=pltpu.VMEM),
        ],
        out_specs=pl.BlockSpec(memory_space=pltpu.VMEM),
        scratch_shapes=[
            pltpu.SemaphoreType.DMA((NMSG,)),
            pltpu.SemaphoreType.DMA((NMSG,)),
        ],
        compiler_params=pltpu.CompilerParams(collective_id=0),
    )(pi, x)


# device time: 32452 ns/iter; 1.8120x vs baseline; 1.0003x over previous
import jax
import jax.numpy as jnp
from jax import lax
from jax.experimental import pallas as pl
from jax.experimental.pallas import tpu as pltpu

QROWS = 256
CH = 32
NCH = QROWS // CH
NMSG = 3 * NCH + 2


def kernel(x, pi):
    def body(pi_ref, x_ref, out_ref, send_sems, recv_sems, local_sem):
        my_x = lax.axis_index("x")
        my_y = lax.axis_index("y")
        my_z = lax.axis_index("z")
        dst_x = pi_ref[my_x]

        @pl.when(dst_x == my_x)
        def _identity():
            cp = pltpu.make_async_copy(x_ref, out_ref, local_sem)
            cp.start()
            cp.wait()

        @pl.when(dst_x != my_x)
        def _swap():
            even = (my_y + my_z) % 2 == 0
            p = jnp.where(my_y == 0, my_z, 3 - my_z)
            cw = (my_x, jnp.where(even, my_y, 1 - my_y),
                  jnp.where(even, 1 - my_z, my_z))
            ccw = (my_x, jnp.where(even, 1 - my_y, my_y),
                   jnp.where(even, my_z, 1 - my_z))
            part = (dst_x, my_y, my_z)

            barrier_sem = pltpu.get_barrier_semaphore()
            for nbr in (part, cw, ccw):
                pl.semaphore_signal(
                    barrier_sem, inc=1, device_id=nbr,
                    device_id_type=pl.DeviceIdType.MESH,
                )
            pl.semaphore_wait(barrier_sem, 3)

            bp = p * QROWS
            b_cw = ((p + 1) % 4) * QROWS
            b_ccw = ((p + 3) % 4) * QROWS

            def rdma(rows, nrows, msg, dev):
                sl = (slice(None), pl.ds(rows, nrows), slice(None))
                src = x_ref.at[sl] if msg < NCH else out_ref.at[sl]
                return pltpu.make_async_remote_copy(
                    src_ref=src,
                    dst_ref=out_ref.at[sl],
                    send_sem=send_sems.at[msg],
                    recv_sem=recv_sems.at[msg],
                    device_id=dev,
                    device_id_type=pl.DeviceIdType.MESH,
                )

            xs = [rdma(bp + k * CH, CH, k, part) for k in range(NCH)]
            for d in xs:
                d.start()

            bcw, bccw = [], []
            for k in range(NCH):
                xs[k].wait_recv()
                d1 = rdma(bp + k * CH, CH, NCH + k, cw)
                d2 = rdma(bp + k * CH, CH, 2 * NCH + k, ccw)
                d1.start()
                d2.start()
                bcw.append(d1)
                bccw.append(d2)

            for k in range(NCH // 2):
                bccw[k].wait_recv()
            f_ccw = rdma(b_cw, QROWS // 2, 3 * NCH + 1, ccw)
            f_ccw.start()
            for k in range(NCH // 2, NCH):
                bcw[k].wait_recv()
            f_cw = rdma(b_ccw + QROWS // 2, QROWS // 2, 3 * NCH, cw)
            f_cw.start()

            for k in range(NCH // 2):
                bcw[k].wait_recv()
            for k in range(NCH // 2, NCH):
                bccw[k].wait_recv()
            f_ccw.wait_recv()
            f_cw.wait_recv()
            for d in xs + bcw + bccw + [f_cw, f_ccw]:
                d.wait_send()

    return pl.pallas_call(
        body,
        out_shape=jax.ShapeDtypeStruct(x.shape, x.dtype),
        in_specs=[
            pl.BlockSpec(memory_space=pltpu.SMEM),
            pl.BlockSpec(memory_space=pl.ANY),
        ],
        out_specs=pl.BlockSpec(memory_space=pl.ANY),
        scratch_shapes=[
            pltpu.SemaphoreType.DMA((NMSG,)),
            pltpu.SemaphoreType.DMA((NMSG,)),
            pltpu.SemaphoreType.DMA,
        ],
        compiler_params=pltpu.CompilerParams(collective_id=0),
    )(pi, x)


# device time: 29692 ns/iter; 1.9804x vs baseline; 1.0930x over previous
import jax
import jax.numpy as jnp
from jax import lax
from jax.experimental import pallas as pl
from jax.experimental.pallas import tpu as pltpu

QROWS = 256
CH = 32
NCH = QROWS // CH
NMSG = 3 * NCH + 3


def kernel(x, pi):
    def body(pi_ref, x_ref, out_ref, send_sems, recv_sems, local_sem):
        my_x = lax.axis_index("x")
        my_y = lax.axis_index("y")
        my_z = lax.axis_index("z")
        dst_x = pi_ref[my_x]

        @pl.when(dst_x == my_x)
        def _identity():
            cp = pltpu.make_async_copy(x_ref, out_ref, local_sem)
            cp.start()
            cp.wait()

        @pl.when(dst_x != my_x)
        def _swap():
            even = (my_y + my_z) % 2 == 0
            p = jnp.where(my_y == 0, my_z, 3 - my_z)
            cw = (my_x, jnp.where(even, my_y, 1 - my_y),
                  jnp.where(even, 1 - my_z, my_z))
            ccw = (my_x, jnp.where(even, 1 - my_y, my_y),
                   jnp.where(even, my_z, 1 - my_z))
            part = (dst_x, my_y, my_z)

            barrier_sem = pltpu.get_barrier_semaphore()
            for nbr in (part, cw, ccw):
                pl.semaphore_signal(
                    barrier_sem, inc=1, device_id=nbr,
                    device_id_type=pl.DeviceIdType.MESH,
                )
            pl.semaphore_wait(barrier_sem, 3)

            bp = p * QROWS
            b_cw = ((p + 1) % 4) * QROWS
            b_ccw = ((p + 3) % 4) * QROWS
            b_opp = ((p + 2) % 4) * QROWS

            def rdma(rows, nrows, msg, dev):
                sl = (slice(None), pl.ds(rows, nrows), slice(None))
                src = x_ref.at[sl] if msg < NCH or msg == 3 * NCH + 2 \
                    else out_ref.at[sl]
                return pltpu.make_async_remote_copy(
                    src_ref=src,
                    dst_ref=out_ref.at[sl],
                    send_sem=send_sems.at[msg],
                    recv_sem=recv_sems.at[msg],
                    device_id=dev,
                    device_id_type=pl.DeviceIdType.MESH,
                )

            xs = [rdma(bp + k * CH, CH, k, part) for k in range(NCH)]
            for d in xs:
                d.start()
            xe = rdma(b_opp, QROWS // 2, 3 * NCH + 2, part)
            xe.start()

            bcw, bccw = [], []
            for k in range(NCH):
                xs[k].wait_recv()
                d1 = rdma(bp + k * CH, CH, NCH + k, cw)
                d2 = rdma(bp + k * CH, CH, 2 * NCH + k, ccw)
                d1.start()
                d2.start()
                bcw.append(d1)
                bccw.append(d2)

            ncut = (QROWS // 2) // CH
            nf = QROWS // 4
            for k in range(ncut, ncut + nf // CH):
                bccw[k].wait_recv()
            f_ccw = rdma(b_cw + QROWS // 2, nf, 3 * NCH + 1, ccw)
            f_ccw.start()
            for k in range(NCH - nf // CH, NCH):
                bcw[k].wait_recv()
            f_cw = rdma(b_ccw + QROWS // 2 + nf, nf, 3 * NCH, cw)
            f_cw.start()

            for k in range(NCH):
                if not (NCH - nf // CH <= k < NCH):
                    bcw[k].wait_recv()
                if not (ncut <= k < ncut + nf // CH):
                    bccw[k].wait_recv()
            xe.wait_recv()
            f_ccw.wait_recv()
            f_cw.wait_recv()
            for d in xs + bcw + bccw + [f_cw, f_ccw, xe]:
                d.wait_send()

    return pl.pallas_call(
        body,
        out_shape=jax.ShapeDtypeStruct(x.shape, x.dtype),
        in_specs=[
            pl.BlockSpec(memory_space=pltpu.SMEM),
            pl.BlockSpec(memory_space=pl.ANY),
        ],
        out_specs=pl.BlockSpec(memory_space=pl.ANY),
        scratch_shapes=[
            pltpu.SemaphoreType.DMA((NMSG,)),
            pltpu.SemaphoreType.DMA((NMSG,)),
            pltpu.SemaphoreType.DMA,
        ],
        compiler_params=pltpu.CompilerParams(collective_id=0),
    )(pi, x)
